# Initial kernel scaffold; baseline (speedup 1.0000x reference)
import jax
import jax.numpy as jnp
from jax.experimental import pallas as pl


def kernel(features, edge_index, batch, W_self, W_neigh, W_cls, b_cls):
    raise NotImplementedError("write your pallas kernel here")


# speedup vs baseline: 5.2130x; 5.2130x over previous
import dataclasses

import jax
import jax.numpy as jnp
from jax import lax
from jax.experimental import pallas as pl
from jax.experimental.pallas import tpu as pltpu
from jax.experimental.pallas import tpu_sc as plsc

N = 10000
E = 320000
D = 128
G = 128
C = 10

NUM_CORES = 2
NUM_SUBCORES = 16
NW = NUM_CORES * NUM_SUBCORES
CHUNK = 128
N_PAD = 10240
ROWS_PER_SUB = N_PAD // NUM_SUBCORES
BLOCKS_PER_SUB = ROWS_PER_SUB // CHUNK
E_PAD = ((E + NW * CHUNK - 1) // (NW * CHUNK)) * (NW * CHUNK)
EPW = E_PAD // NW
NCHUNK = EPW // CHUNK
LANES = 16


def _sc_body(src_hbm, dst_hbm, feat_hbm, zf_hbm, zd_hbm,
             agg_out, deg_out, idx_src_v, idx_dst_v, rows_v, deg_v,
             agg_sh, sem):
    c = lax.axis_index("c")
    s = lax.axis_index("s")
    wid = c * NUM_SUBCORES + s
    r0 = s * ROWS_PER_SUB
    out0 = c * N_PAD + r0

    pltpu.sync_copy(zf_hbm, rows_v)
    pltpu.sync_copy(zd_hbm, deg_v)

    @pl.loop(0, BLOCKS_PER_SUB)
    def _(b):
        pltpu.sync_copy(rows_v, agg_sh.at[pl.ds(r0 + b * CHUNK, CHUNK)])

    plsc.subcore_barrier()

    ones16 = jnp.ones((LANES,), jnp.float32)

    @pl.loop(0, NCHUNK)
    def _(g):
        base = wid * EPW + g * CHUNK
        pltpu.sync_copy(src_hbm.at[pl.ds(base, CHUNK)], idx_src_v)
        pltpu.sync_copy(dst_hbm.at[pl.ds(base, CHUNK)], idx_dst_v)
        pltpu.async_copy(feat_hbm.at[idx_src_v], rows_v, sem).wait()
        pltpu.sync_copy(rows_v, agg_sh.at[idx_dst_v], add=True)

        @pl.loop(0, CHUNK, step=LANES)
        def _(j):
            idx16 = idx_dst_v[pl.ds(j, LANES)]
            plsc.addupdate_scatter(deg_v, [idx16], ones16)

    plsc.subcore_barrier()

    @pl.loop(0, BLOCKS_PER_SUB)
    def _(b):
        pltpu.sync_copy(agg_sh.at[pl.ds(r0 + b * CHUNK, CHUNK)], rows_v)
        pltpu.sync_copy(rows_v, agg_out.at[pl.ds(out0 + b * CHUNK, CHUNK)])

    pltpu.sync_copy(deg_v, deg_out.at[pl.ds(wid * N_PAD, N_PAD)])


def _sc_segment_sum(src_pad, dst_pad, features, zeros_feat, zeros_deg):
    mesh = plsc.VectorSubcoreMesh(core_axis_name="c", subcore_axis_name="s")
    cp = pltpu.CompilerParams()
    if "needs_layout_passes" in pltpu.CompilerParams.__dataclass_fields__:
        cp = dataclasses.replace(cp, needs_layout_passes=False)
    kern = pl.kernel(
        _sc_body,
        compiler_params=cp,
        out_type=(
            jax.ShapeDtypeStruct((NUM_CORES * N_PAD, D), jnp.float32),
            jax.ShapeDtypeStruct((NW * N_PAD,), jnp.float32),
        ),
        mesh=mesh,
        scratch_types=[
            pltpu.VMEM((CHUNK,), jnp.int32),
            pltpu.VMEM((CHUNK,), jnp.int32),
            pltpu.VMEM((CHUNK, D), jnp.float32),
            pltpu.VMEM((N_PAD,), jnp.float32),
            pltpu.VMEM_SHARED((N_PAD, D), jnp.float32),
            pltpu.SemaphoreType.DMA,
        ],
    )
    return kern(src_pad, dst_pad, features, zeros_feat, zeros_deg)


def _tc_body(feat, agg2, degw, batch_row, ws, wn, wc, bc,
             scores_out, gemb_out):
    agg = agg2[0:N, :] + agg2[N_PAD:N_PAD + N, :]
    deg = jnp.sum(degw[0:N, :], axis=1, keepdims=True)
    mean = agg / jnp.maximum(deg, 1.0)
    h = jnp.dot(feat[...], ws[...], preferred_element_type=jnp.float32)
    h = h + jnp.dot(mean, wn[...], preferred_element_type=jnp.float32)
    emb = jnp.maximum(h, 0.0)
    iota_g = lax.broadcasted_iota(jnp.int32, (G, N), 0)
    onehot_t = (batch_row[...] == iota_g).astype(jnp.float32)
    gemb = jnp.dot(onehot_t, emb, preferred_element_type=jnp.float32)
    gemb_out[...] = gemb
    scores_out[...] = jnp.dot(gemb, wc[...],
                              preferred_element_type=jnp.float32) + bc[...]


def kernel(features, edge_index, batch, W_self, W_neigh, W_cls, b_cls):
    src = edge_index[0]
    dst = edge_index[1]
    pad = E_PAD - E
    src_pad = jnp.concatenate([src, jnp.zeros((pad,), src.dtype)])
    dst_pad = jnp.concatenate([dst, jnp.full((pad,), N, dst.dtype)])
    zeros_feat = jnp.zeros((CHUNK, D), jnp.float32)
    zeros_deg = jnp.zeros((N_PAD,), jnp.float32)

    agg2, degw = _sc_segment_sum(src_pad.astype(jnp.int32),
                                 dst_pad.astype(jnp.int32),
                                 features, zeros_feat, zeros_deg)

    batch_row = batch.astype(jnp.int32).reshape(1, N)
    scores, gemb = pl.pallas_call(
        _tc_body,
        out_shape=(
            jax.ShapeDtypeStruct((G, C), jnp.float32),
            jax.ShapeDtypeStruct((G, D), jnp.float32),
        ),
    )(features, agg2, degw.reshape(NW, N_PAD).T, batch_row, W_self, W_neigh,
      W_cls, b_cls.reshape(1, C))
    return (scores, gemb)
